# Initial kernel scaffold; baseline (speedup 1.0000x reference)
import jax
import jax.numpy as jnp
from jax.experimental import pallas as pl


def kernel(x, table, W1, b1, W2, b2):
    raise NotImplementedError("write your pallas kernel here")


# speedup vs baseline: 2.9345x; 2.9345x over previous
import functools

import jax
import jax.numpy as jnp
from jax import lax
from jax.experimental import pallas as pl
from jax.experimental.pallas import tpu as pltpu
from jax.experimental.pallas import tpu_sc as plsc

B = 16384
H = 200
D = 64
NC = 2
NS = 16
NW = NC * NS
BPW = B // NW
CRI = 32
NCH = BPW // CRI
H1, H2 = 104, 96

_INV_H = 1.0 / float(H)


def _pool_sc(x, table):
    mesh = plsc.VectorSubcoreMesh(core_axis_name="c", subcore_axis_name="s")

    @functools.partial(
        pl.kernel,
        out_type=jax.ShapeDtypeStruct((B, D), jnp.float32),
        mesh=mesh,
        compiler_params=pltpu.CompilerParams(use_tc_tiling_on_sc=False),
        scratch_types=[
            pltpu.VMEM((2, CRI, H), jnp.int32),
            pltpu.VMEM((2, H, D), jnp.float32),
            pltpu.VMEM((BPW, D), jnp.float32),
            pltpu.SemaphoreType.DMA,
            pltpu.SemaphoreType.DMA,
            pltpu.SemaphoreType.DMA,
        ],
    )
    def kern(x_hbm, tab_hbm, out_hbm, idx_v, rows_v, pool_v, sem0, sem1, semi):
        wid = lax.axis_index("s") * NC + lax.axis_index("c")
        base = wid * BPW

        def start_gather(buf, sem, cbuf, rr):
            pltpu.async_copy(
                tab_hbm.at[idx_v.at[cbuf, rr, pl.ds(0, H1)]],
                rows_v.at[buf, pl.ds(0, H1), :], sem)
            pltpu.async_copy(
                tab_hbm.at[idx_v.at[cbuf, rr, pl.ds(H1, H2)]],
                rows_v.at[buf, pl.ds(H1, H2), :], sem)

        def wait_gather(buf, sem):
            pltpu.make_async_copy(
                tab_hbm.at[pl.ds(0, H), :], rows_v.at[buf], sem).wait()

        def wait_idx():
            pltpu.make_async_copy(
                x_hbm.at[pl.ds(0, CRI), :], idx_v.at[0], semi).wait()

        def accumulate(buf, row):
            def acc_body(j, acc):
                a0, a1, a2, a3 = acc
                for u in range(8):
                    jj = j * 8 + u
                    a0 = a0 + rows_v[buf, jj, pl.ds(0, 16)]
                    a1 = a1 + rows_v[buf, jj, pl.ds(16, 16)]
                    a2 = a2 + rows_v[buf, jj, pl.ds(32, 16)]
                    a3 = a3 + rows_v[buf, jj, pl.ds(48, 16)]
                return (a0, a1, a2, a3)

            zero = jnp.zeros((16,), jnp.float32)
            a0, a1, a2, a3 = lax.fori_loop(
                0, H // 8, acc_body, (zero, zero, zero, zero))
            pool_v[row, pl.ds(0, 16)] = a0 * _INV_H
            pool_v[row, pl.ds(16, 16)] = a1 * _INV_H
            pool_v[row, pl.ds(32, 16)] = a2 * _INV_H
            pool_v[row, pl.ds(48, 16)] = a3 * _INV_H

        pltpu.async_copy(
            x_hbm.at[pl.ds(base, CRI), :], idx_v.at[0], semi).wait()
        pltpu.async_copy(
            x_hbm.at[pl.ds(base + CRI, CRI), :], idx_v.at[1], semi)
        start_gather(0, sem0, 0, 0)
        start_gather(1, sem1, 0, 1)

        def body(i, _):
            r0 = 2 * i

            wait_gather(0, sem0)
            accumulate(0, r0)
            nxt = r0 + 2

            @pl.when(nxt < BPW)
            def _():
                c_nxt = nxt // CRI
                rr = lax.rem(nxt, CRI)

                @pl.when(rr == 0)
                def _():
                    wait_idx()

                    @pl.when(c_nxt + 1 < NCH)
                    def _():
                        pltpu.async_copy(
                            x_hbm.at[pl.ds(base + (c_nxt + 1) * CRI, CRI), :],
                            idx_v.at[lax.rem(c_nxt + 1, 2)], semi)

                start_gather(0, sem0, lax.rem(c_nxt, 2), rr)

            wait_gather(1, sem1)
            accumulate(1, r0 + 1)
            nxt1 = r0 + 3

            @pl.when(nxt1 < BPW)
            def _():
                c_nxt1 = nxt1 // CRI
                start_gather(1, sem1, lax.rem(c_nxt1, 2), lax.rem(nxt1, CRI))

            return 0

        lax.fori_loop(0, BPW // 2, body, 0)
        pltpu.sync_copy(pool_v, out_hbm.at[pl.ds(base, BPW), :])

    return kern(x, table)


BLK = 1024
NOUT_PAD = 128


def _mlp_body(p_ref, w1_ref, b1_ref, w2_ref, b2_ref, o_ref):
    h = jnp.dot(p_ref[:], w1_ref[:], preferred_element_type=jnp.float32)
    h = jnp.maximum(h + b1_ref[:], 0.0)
    out = jnp.dot(h, w2_ref[:], preferred_element_type=jnp.float32)
    out = out + b2_ref[:]
    nrm = jnp.sqrt(jnp.sum(out * out, axis=-1, keepdims=True))
    o_ref[:] = out / jnp.maximum(nrm, 1e-12)


def _mlp_tc(pooled, W1, b1r, W2p, b2p):
    return pl.pallas_call(
        _mlp_body,
        grid=(B // BLK,),
        in_specs=[
            pl.BlockSpec((BLK, D), lambda i: (i, 0)),
            pl.BlockSpec((D, D), lambda i: (0, 0)),
            pl.BlockSpec((1, D), lambda i: (0, 0)),
            pl.BlockSpec((D, NOUT_PAD), lambda i: (0, 0)),
            pl.BlockSpec((1, NOUT_PAD), lambda i: (0, 0)),
        ],
        out_specs=pl.BlockSpec((BLK, NOUT_PAD), lambda i: (i, 0)),
        out_shape=jax.ShapeDtypeStruct((B, NOUT_PAD), jnp.float32),
    )(pooled, W1, b1r, W2p, b2p)


@jax.jit
def kernel(x, table, W1, b1, W2, b2):
    x = x.astype(jnp.int32)
    pooled = _pool_sc(x, table)
    nout = W2.shape[1]
    W2p = jnp.pad(W2, ((0, 0), (0, NOUT_PAD - nout)))
    b2p = jnp.pad(b2, (0, NOUT_PAD - nout)).reshape(1, NOUT_PAD)
    out = _mlp_tc(pooled, W1, b1.reshape(1, D), W2p, b2p)
    return out[:, :nout]


# speedup vs baseline: 3.2817x; 1.1183x over previous
import functools

import jax
import jax.numpy as jnp
from jax import lax
from jax.experimental import pallas as pl
from jax.experimental.pallas import tpu as pltpu
from jax.experimental.pallas import tpu_sc as plsc

B = 16384
H = 200
D = 64
V = 1_000_000
NC = 2
NS = 16
NW = NC * NS
BPW = B // NW
CRI = 32
NCH = BPW // CRI
H1, H2 = 104, 96

BN = 1024
GA = 489
S = GA * BN
V2 = 2 * S


def _pack_body(t1_ref, t2_ref, w_ref, o_ref):
    o_ref[:, 0:D] = lax.dot_general(
        t1_ref[:], w_ref[:], (((0,), (0,)), ((), ())),
        preferred_element_type=jnp.float32)
    o_ref[:, D:2 * D] = lax.dot_general(
        t2_ref[:], w_ref[:], (((0,), (0,)), ((), ())),
        preferred_element_type=jnp.float32)


def _pack_tc(tabT, W1s):
    return pl.pallas_call(
        _pack_body,
        grid=(GA,),
        in_specs=[
            pl.BlockSpec((D, BN), lambda i: (0, i)),
            pl.BlockSpec((D, BN), lambda i: (0, jnp.minimum(i + GA, (V + BN - 1) // BN - 1))),
            pl.BlockSpec((D, D), lambda i: (0, 0)),
        ],
        out_specs=pl.BlockSpec((BN, 2 * D), lambda i: (i, 0)),
        out_shape=jax.ShapeDtypeStruct((S, 2 * D), jnp.float32),
    )(tabT, tabT, W1s)


def _pool_sc(xf, tab):
    mesh = plsc.VectorSubcoreMesh(core_axis_name="c", subcore_axis_name="s")

    @functools.partial(
        pl.kernel,
        out_type=jax.ShapeDtypeStruct((B, D), jnp.float32),
        mesh=mesh,
        compiler_params=pltpu.CompilerParams(use_tc_tiling_on_sc=False),
        scratch_types=[
            pltpu.VMEM((2 * CRI * H,), jnp.int32),
            pltpu.VMEM((2, H, D), jnp.float32),
            pltpu.VMEM((BPW, D), jnp.float32),
            pltpu.SemaphoreType.DMA,
            pltpu.SemaphoreType.DMA,
            pltpu.SemaphoreType.DMA,
        ],
    )
    def kern(x_hbm, tab_hbm, out_hbm, idx_v, rows_v, pool_v, sem0, sem1, semi):
        wid = lax.axis_index("s") * NC + lax.axis_index("c")
        base = wid * BPW

        def start_gather(buf, sem, cbuf, rr):
            off = cbuf * CRI * H + rr * H
            pltpu.async_copy(
                tab_hbm.at[idx_v.at[pl.ds(off, H1)]],
                rows_v.at[buf, pl.ds(0, H1), :], sem)
            pltpu.async_copy(
                tab_hbm.at[idx_v.at[pl.ds(off + H1, H2)]],
                rows_v.at[buf, pl.ds(H1, H2), :], sem)

        def wait_gather(buf, sem):
            pltpu.make_async_copy(
                tab_hbm.at[pl.ds(0, H), :], rows_v.at[buf], sem).wait()

        def wait_idx():
            pltpu.make_async_copy(
                x_hbm.at[pl.ds(0, CRI * H)], idx_v.at[pl.ds(0, CRI * H)],
                semi).wait()

        def accumulate(buf, row):
            def acc_body(j, acc):
                a0, a1, a2, a3 = acc
                for u in range(8):
                    jj = j * 8 + u
                    a0 = a0 + rows_v[buf, jj, pl.ds(0, 16)]
                    a1 = a1 + rows_v[buf, jj, pl.ds(16, 16)]
                    a2 = a2 + rows_v[buf, jj, pl.ds(32, 16)]
                    a3 = a3 + rows_v[buf, jj, pl.ds(48, 16)]
                return (a0, a1, a2, a3)

            zero = jnp.zeros((16,), jnp.float32)
            a0, a1, a2, a3 = lax.fori_loop(
                0, H // 8, acc_body, (zero, zero, zero, zero))
            pool_v[row, pl.ds(0, 16)] = a0
            pool_v[row, pl.ds(16, 16)] = a1
            pool_v[row, pl.ds(32, 16)] = a2
            pool_v[row, pl.ds(48, 16)] = a3

        pltpu.async_copy(
            x_hbm.at[pl.ds(base * H, CRI * H)],
            idx_v.at[pl.ds(0, CRI * H)], semi).wait()
        pltpu.async_copy(
            x_hbm.at[pl.ds((base + CRI) * H, CRI * H)],
            idx_v.at[pl.ds(CRI * H, CRI * H)], semi)
        start_gather(0, sem0, 0, 0)
        start_gather(1, sem1, 0, 1)

        def body(i, _):
            r0 = 2 * i

            wait_gather(0, sem0)
            accumulate(0, r0)
            nxt = r0 + 2

            @pl.when(nxt < BPW)
            def _():
                c_nxt = nxt // CRI
                rr = lax.rem(nxt, CRI)

                @pl.when(rr == 0)
                def _():
                    wait_idx()

                    @pl.when(c_nxt + 1 < NCH)
                    def _():
                        pltpu.async_copy(
                            x_hbm.at[pl.ds((base + (c_nxt + 1) * CRI) * H,
                                           CRI * H)],
                            idx_v.at[pl.ds(lax.rem(c_nxt + 1, 2) * CRI * H,
                                           CRI * H)], semi)

                start_gather(0, sem0, lax.rem(c_nxt, 2), rr)

            wait_gather(1, sem1)
            accumulate(1, r0 + 1)
            nxt1 = r0 + 3

            @pl.when(nxt1 < BPW)
            def _():
                c_nxt1 = nxt1 // CRI
                start_gather(1, sem1, lax.rem(c_nxt1, 2), lax.rem(nxt1, CRI))

            return 0

        lax.fori_loop(0, BPW // 2, body, 0)
        pltpu.sync_copy(pool_v, out_hbm.at[pl.ds(base, BPW), :])

    return kern(xf, tab)


BLK = 1024
NOUT_PAD = 128


def _mlp_body(p_ref, b1_ref, w2_ref, b2_ref, o_ref):
    h = jnp.maximum(p_ref[:] + b1_ref[:], 0.0)
    out = jnp.dot(h, w2_ref[:], preferred_element_type=jnp.float32)
    out = out + b2_ref[:]
    nrm = jnp.sqrt(jnp.sum(out * out, axis=-1, keepdims=True))
    o_ref[:] = out / jnp.maximum(nrm, 1e-12)


def _mlp_tc(pooled, b1r, W2p, b2p):
    return pl.pallas_call(
        _mlp_body,
        grid=(B // BLK,),
        in_specs=[
            pl.BlockSpec((BLK, D), lambda i: (i, 0)),
            pl.BlockSpec((1, D), lambda i: (0, 0)),
            pl.BlockSpec((D, NOUT_PAD), lambda i: (0, 0)),
            pl.BlockSpec((1, NOUT_PAD), lambda i: (0, 0)),
        ],
        out_specs=pl.BlockSpec((BLK, NOUT_PAD), lambda i: (i, 0)),
        out_shape=jax.ShapeDtypeStruct((B, NOUT_PAD), jnp.float32),
    )(pooled, b1r, W2p, b2p)


@jax.jit
def kernel(x, table, W1, b1, W2, b2):
    x = x.astype(jnp.int32)
    packed = _pack_tc(table.T, W1 * (1.0 / float(H)))
    tab2 = packed.reshape(V2, D)
    xr = jnp.where(x < S, 2 * x, 2 * x - (2 * S - 1)).reshape(-1)
    pooled = _pool_sc(xr, tab2)
    nout = W2.shape[1]
    W2p = jnp.pad(W2, ((0, 0), (0, NOUT_PAD - nout)))
    b2p = jnp.pad(b2, (0, NOUT_PAD - nout)).reshape(1, NOUT_PAD)
    out = _mlp_tc(pooled, b1.reshape(1, D), W2p, b2p)
    return out[:, :nout]


# speedup vs baseline: 4.1518x; 1.2651x over previous
import functools

import jax
import jax.numpy as jnp
from jax import lax
from jax.experimental import pallas as pl
from jax.experimental.pallas import tpu as pltpu
from jax.experimental.pallas import tpu_sc as plsc

B = 16384
H = 200
D = 64
V = 1_000_000
NC = 2
NS = 16
NW = NC * NS
BPW = B // NW
CRI = 32
NCH = BPW // CRI
H1, H2 = 104, 96

BN = 1024
CPB = 4 * BN
GA = (V + CPB - 1) // CPB
S4 = GA * BN
V2 = 4 * S4
WPR = D // 2

_PERM = (
    list(range(0, 16)) + list(range(32, 48))
    + list(range(16, 32)) + list(range(48, 64))
)


def _bf16_bits(x):
    u = lax.bitcast_convert_type(x, jnp.uint32)
    r = u + jnp.uint32(0x7FFF) + ((u >> 16) & jnp.uint32(1))
    return r & jnp.uint32(0xFFFF0000)


def _pack_body(t_ref, w_ref, o_ref):
    for k in range(4):
        rk = lax.dot_general(
            t_ref[:, k * BN:(k + 1) * BN], w_ref[:],
            (((0,), (0,)), ((), ())), preferred_element_type=jnp.float32)
        lo = _bf16_bits(rk[:, 0:WPR]) >> 16
        hi = _bf16_bits(rk[:, WPR:D])
        o_ref[:, WPR * k:WPR * (k + 1)] = lo | hi


def _pack_tc(tabT, W1s):
    return pl.pallas_call(
        _pack_body,
        grid=(GA,),
        in_specs=[
            pl.BlockSpec((D, CPB), lambda i: (0, i)),
            pl.BlockSpec((D, D), lambda i: (0, 0)),
        ],
        out_specs=pl.BlockSpec((BN, 4 * WPR), lambda i: (i, 0)),
        out_shape=jax.ShapeDtypeStruct((S4, 4 * WPR), jnp.uint32),
    )(tabT, W1s)


def _pool_sc(xf, tab):
    mesh = plsc.VectorSubcoreMesh(core_axis_name="c", subcore_axis_name="s")

    @functools.partial(
        pl.kernel,
        out_type=jax.ShapeDtypeStruct((B, D), jnp.float32),
        mesh=mesh,
        compiler_params=pltpu.CompilerParams(
            use_tc_tiling_on_sc=False, needs_layout_passes=False),
        scratch_types=[
            pltpu.VMEM((2 * CRI * H,), jnp.int32),
            pltpu.VMEM((2, H, WPR), jnp.uint32),
            pltpu.VMEM((BPW, D), jnp.float32),
            pltpu.SemaphoreType.DMA,
            pltpu.SemaphoreType.DMA,
            pltpu.SemaphoreType.DMA,
        ],
    )
    def kern(x_hbm, tab_hbm, out_hbm, idx_v, rows_v, pool_v, sem0, sem1, semi):
        wid = lax.axis_index("s") * NC + lax.axis_index("c")
        base = wid * BPW

        def start_gather(buf, sem, cbuf, rr):
            off = cbuf * CRI * H + rr * H
            pltpu.async_copy(
                tab_hbm.at[idx_v.at[pl.ds(off, H1)]],
                rows_v.at[buf, pl.ds(0, H1), :], sem)
            pltpu.async_copy(
                tab_hbm.at[idx_v.at[pl.ds(off + H1, H2)]],
                rows_v.at[buf, pl.ds(H1, H2), :], sem)

        def wait_gather(buf, sem):
            pltpu.make_async_copy(
                tab_hbm.at[pl.ds(0, H), :], rows_v.at[buf], sem).wait()

        def wait_idx():
            pltpu.make_async_copy(
                x_hbm.at[pl.ds(0, CRI * H)], idx_v.at[pl.ds(0, CRI * H)],
                semi).wait()

        def accumulate(buf, row):
            def acc_body(j, acc):
                a0, a1, a2, a3 = acc
                for u in range(4):
                    jj = j * 4 + u
                    w0 = rows_v[buf, jj, pl.ds(0, 16)]
                    w1 = rows_v[buf, jj, pl.ds(16, 16)]
                    p0, q0 = plsc.unpack(
                        plsc.bitcast(w0, jnp.bfloat16),
                        format=plsc.PackFormat.INTERLEAVED,
                        preferred_element_type=jnp.float32)
                    p1, q1 = plsc.unpack(
                        plsc.bitcast(w1, jnp.bfloat16),
                        format=plsc.PackFormat.INTERLEAVED,
                        preferred_element_type=jnp.float32)
                    a0 = a0 + p0
                    a1 = a1 + q0
                    a2 = a2 + p1
                    a3 = a3 + q1
                return (a0, a1, a2, a3)

            zero = jnp.zeros((16,), jnp.float32)
            a0, a1, a2, a3 = lax.fori_loop(
                0, H // 4, acc_body, (zero, zero, zero, zero))
            pool_v[row, pl.ds(0, 16)] = a0
            pool_v[row, pl.ds(16, 16)] = a1
            pool_v[row, pl.ds(32, 16)] = a2
            pool_v[row, pl.ds(48, 16)] = a3

        pltpu.async_copy(
            x_hbm.at[pl.ds(base * H, CRI * H)],
            idx_v.at[pl.ds(0, CRI * H)], semi).wait()
        pltpu.async_copy(
            x_hbm.at[pl.ds((base + CRI) * H, CRI * H)],
            idx_v.at[pl.ds(CRI * H, CRI * H)], semi)
        start_gather(0, sem0, 0, 0)
        start_gather(1, sem1, 0, 1)

        def body(i, _):
            r0 = 2 * i

            wait_gather(0, sem0)
            accumulate(0, r0)
            nxt = r0 + 2

            @pl.when(nxt < BPW)
            def _():
                c_nxt = nxt // CRI
                rr = lax.rem(nxt, CRI)

                @pl.when(rr == 0)
                def _():
                    wait_idx()

                    @pl.when(c_nxt + 1 < NCH)
                    def _():
                        pltpu.async_copy(
                            x_hbm.at[pl.ds((base + (c_nxt + 1) * CRI) * H,
                                           CRI * H)],
                            idx_v.at[pl.ds(lax.rem(c_nxt + 1, 2) * CRI * H,
                                           CRI * H)], semi)

                start_gather(0, sem0, lax.rem(c_nxt, 2), rr)

            wait_gather(1, sem1)
            accumulate(1, r0 + 1)
            nxt1 = r0 + 3

            @pl.when(nxt1 < BPW)
            def _():
                c_nxt1 = nxt1 // CRI
                start_gather(1, sem1, lax.rem(c_nxt1, 2), lax.rem(nxt1, CRI))

            return 0

        lax.fori_loop(0, BPW // 2, body, 0)
        pltpu.sync_copy(pool_v, out_hbm.at[pl.ds(base, BPW), :])

    return kern(xf, tab)


BLK = 1024
NOUT_PAD = 128


def _mlp_body(p_ref, b1_ref, w2_ref, b2_ref, o_ref):
    h = jnp.maximum(p_ref[:] + b1_ref[:], 0.0)
    out = jnp.dot(h, w2_ref[:], preferred_element_type=jnp.float32)
    out = out + b2_ref[:]
    nrm = jnp.sqrt(jnp.sum(out * out, axis=-1, keepdims=True))
    o_ref[:] = out / jnp.maximum(nrm, 1e-12)


def _mlp_tc(pooled, b1r, W2p, b2p):
    return pl.pallas_call(
        _mlp_body,
        grid=(B // BLK,),
        in_specs=[
            pl.BlockSpec((BLK, D), lambda i: (i, 0)),
            pl.BlockSpec((1, D), lambda i: (0, 0)),
            pl.BlockSpec((D, NOUT_PAD), lambda i: (0, 0)),
            pl.BlockSpec((1, NOUT_PAD), lambda i: (0, 0)),
        ],
        out_specs=pl.BlockSpec((BLK, NOUT_PAD), lambda i: (i, 0)),
        out_shape=jax.ShapeDtypeStruct((B, NOUT_PAD), jnp.float32),
    )(pooled, b1r, W2p, b2p)


@jax.jit
def kernel(x, table, W1, b1, W2, b2):
    x = x.astype(jnp.int32)
    packed = _pack_tc(table.T, W1 * (1.0 / float(H)))
    tab2 = packed.reshape(V2, WPR)
    xr = ((x & ~4095) | ((x & 1023) << 2) | ((x >> 10) & 3)).reshape(-1)
    pooled = _pool_sc(xr, tab2)
    perm = jnp.asarray(_PERM, dtype=jnp.int32)
    nout = W2.shape[1]
    W2perm = W2[perm, :]
    W2p = jnp.pad(W2perm, ((0, 0), (0, NOUT_PAD - nout)))
    b2p = jnp.pad(b2, (0, NOUT_PAD - nout)).reshape(1, NOUT_PAD)
    out = _mlp_tc(pooled, b1[perm].reshape(1, D), W2p, b2p)
    return out[:, :nout]


# speedup vs baseline: 4.1521x; 1.0001x over previous
import functools

import jax
import jax.numpy as jnp
from jax import lax
from jax.experimental import pallas as pl
from jax.experimental.pallas import tpu as pltpu
from jax.experimental.pallas import tpu_sc as plsc

B = 16384
H = 200
D = 64
V = 1_000_000
NC = 2
NS = 16
NW = NC * NS
BPW = B // NW
CRI = 32
NCH = BPW // CRI
H1, H2 = 104, 96

BN = 1024
CPB = 4 * BN
GA = (V + CPB - 1) // CPB
S4 = GA * BN
V2 = 4 * S4
WPR = D // 2

_PERM = (
    list(range(0, 16)) + list(range(32, 48))
    + list(range(16, 32)) + list(range(48, 64))
)


def _bf16_bits(x):
    u = lax.bitcast_convert_type(x, jnp.uint32)
    r = u + jnp.uint32(0x7FFF) + ((u >> 16) & jnp.uint32(1))
    return r & jnp.uint32(0xFFFF0000)


def _pack_body(t_ref, w_ref, o_ref):
    for k in range(4):
        rk = lax.dot_general(
            t_ref[:, k * BN:(k + 1) * BN], w_ref[:],
            (((0,), (0,)), ((), ())), preferred_element_type=jnp.float32)
        lo = _bf16_bits(rk[:, 0:WPR]) >> 16
        hi = _bf16_bits(rk[:, WPR:D])
        o_ref[:, WPR * k:WPR * (k + 1)] = lo | hi


def _pack_tc(tabT, W1s):
    return pl.pallas_call(
        _pack_body,
        grid=(GA,),
        in_specs=[
            pl.BlockSpec((D, CPB), lambda i: (0, i)),
            pl.BlockSpec((D, D), lambda i: (0, 0)),
        ],
        out_specs=pl.BlockSpec((BN, 4 * WPR), lambda i: (i, 0)),
        out_shape=jax.ShapeDtypeStruct((S4, 4 * WPR), jnp.uint32),
    )(tabT, W1s)


def _pool_sc(xf, tab):
    mesh = plsc.VectorSubcoreMesh(core_axis_name="c", subcore_axis_name="s")

    @functools.partial(
        pl.kernel,
        out_type=jax.ShapeDtypeStruct((B, D), jnp.float32),
        mesh=mesh,
        compiler_params=pltpu.CompilerParams(
            use_tc_tiling_on_sc=False, needs_layout_passes=False),
        scratch_types=[
            pltpu.VMEM((2 * CRI * H,), jnp.int32),
            pltpu.VMEM((2, H, WPR), jnp.uint32),
            pltpu.VMEM((BPW, D), jnp.float32),
            pltpu.SemaphoreType.DMA,
            pltpu.SemaphoreType.DMA,
            pltpu.SemaphoreType.DMA,
        ],
    )
    def kern(x_hbm, tab_hbm, out_hbm, idx_v, rows_v, pool_v, sem0, sem1, semi):
        wid = lax.axis_index("s") * NC + lax.axis_index("c")
        base = wid * BPW

        def start_gather(buf, sem, cbuf, rr):
            off = cbuf * CRI * H + rr * H
            pltpu.async_copy(
                tab_hbm.at[idx_v.at[pl.ds(off, H1)]],
                rows_v.at[buf, pl.ds(0, H1), :], sem)
            pltpu.async_copy(
                tab_hbm.at[idx_v.at[pl.ds(off + H1, H2)]],
                rows_v.at[buf, pl.ds(H1, H2), :], sem)

        def wait_gather(buf, sem):
            pltpu.make_async_copy(
                tab_hbm.at[pl.ds(0, H), :], rows_v.at[buf], sem).wait()

        def wait_idx():
            pltpu.make_async_copy(
                x_hbm.at[pl.ds(0, CRI * H)], idx_v.at[pl.ds(0, CRI * H)],
                semi).wait()

        hi_mask = jnp.full((16,), 0xFFFF0000, dtype=jnp.uint32)

        def accumulate(buf, row):
            def acc_body(j, acc):
                a0, a1, a2, a3 = acc
                for u in range(8):
                    jj = j * 8 + u
                    w0 = rows_v[buf, jj, pl.ds(0, 16)]
                    w1 = rows_v[buf, jj, pl.ds(16, 16)]
                    a0 = a0 + lax.bitcast_convert_type(
                        w0 << 16, jnp.float32)
                    a1 = a1 + lax.bitcast_convert_type(
                        w0 & hi_mask, jnp.float32)
                    a2 = a2 + lax.bitcast_convert_type(
                        w1 << 16, jnp.float32)
                    a3 = a3 + lax.bitcast_convert_type(
                        w1 & hi_mask, jnp.float32)
                return (a0, a1, a2, a3)

            zero = jnp.zeros((16,), jnp.float32)
            a0, a1, a2, a3 = lax.fori_loop(
                0, H // 8, acc_body, (zero, zero, zero, zero))
            pool_v[row, pl.ds(0, 16)] = a0
            pool_v[row, pl.ds(16, 16)] = a1
            pool_v[row, pl.ds(32, 16)] = a2
            pool_v[row, pl.ds(48, 16)] = a3

        pltpu.async_copy(
            x_hbm.at[pl.ds(base * H, CRI * H)],
            idx_v.at[pl.ds(0, CRI * H)], semi).wait()
        pltpu.async_copy(
            x_hbm.at[pl.ds((base + CRI) * H, CRI * H)],
            idx_v.at[pl.ds(CRI * H, CRI * H)], semi)
        start_gather(0, sem0, 0, 0)
        start_gather(1, sem1, 0, 1)

        def body(i, _):
            r0 = 2 * i

            wait_gather(0, sem0)
            accumulate(0, r0)
            nxt = r0 + 2

            @pl.when(nxt < BPW)
            def _():
                c_nxt = nxt // CRI
                rr = lax.rem(nxt, CRI)

                @pl.when(rr == 0)
                def _():
                    wait_idx()

                    @pl.when(c_nxt + 1 < NCH)
                    def _():
                        pltpu.async_copy(
                            x_hbm.at[pl.ds((base + (c_nxt + 1) * CRI) * H,
                                           CRI * H)],
                            idx_v.at[pl.ds(lax.rem(c_nxt + 1, 2) * CRI * H,
                                           CRI * H)], semi)

                start_gather(0, sem0, lax.rem(c_nxt, 2), rr)

            wait_gather(1, sem1)
            accumulate(1, r0 + 1)
            nxt1 = r0 + 3

            @pl.when(nxt1 < BPW)
            def _():
                c_nxt1 = nxt1 // CRI
                start_gather(1, sem1, lax.rem(c_nxt1, 2), lax.rem(nxt1, CRI))

            return 0

        lax.fori_loop(0, BPW // 2, body, 0)
        pltpu.sync_copy(pool_v, out_hbm.at[pl.ds(base, BPW), :])

    return kern(xf, tab)


BLK = 1024
NOUT_PAD = 128


def _mlp_body(p_ref, b1_ref, w2_ref, b2_ref, o_ref):
    h = jnp.maximum(p_ref[:] + b1_ref[:], 0.0)
    out = jnp.dot(h, w2_ref[:], preferred_element_type=jnp.float32)
    out = out + b2_ref[:]
    nrm = jnp.sqrt(jnp.sum(out * out, axis=-1, keepdims=True))
    o_ref[:] = out / jnp.maximum(nrm, 1e-12)


def _mlp_tc(pooled, b1r, W2p, b2p):
    return pl.pallas_call(
        _mlp_body,
        grid=(B // BLK,),
        in_specs=[
            pl.BlockSpec((BLK, D), lambda i: (i, 0)),
            pl.BlockSpec((1, D), lambda i: (0, 0)),
            pl.BlockSpec((D, NOUT_PAD), lambda i: (0, 0)),
            pl.BlockSpec((1, NOUT_PAD), lambda i: (0, 0)),
        ],
        out_specs=pl.BlockSpec((BLK, NOUT_PAD), lambda i: (i, 0)),
        out_shape=jax.ShapeDtypeStruct((B, NOUT_PAD), jnp.float32),
    )(pooled, b1r, W2p, b2p)


@jax.jit
def kernel(x, table, W1, b1, W2, b2):
    x = x.astype(jnp.int32)
    packed = _pack_tc(table.T, W1 * (1.0 / float(H)))
    tab2 = packed.reshape(V2, WPR)
    xr = ((x & ~4095) | ((x & 1023) << 2) | ((x >> 10) & 3)).reshape(-1)
    pooled = _pool_sc(xr, tab2)
    perm = jnp.asarray(_PERM, dtype=jnp.int32)
    nout = W2.shape[1]
    W2perm = W2[perm, :]
    W2p = jnp.pad(W2perm, ((0, 0), (0, NOUT_PAD - nout)))
    b2p = jnp.pad(b2, (0, NOUT_PAD - nout)).reshape(1, NOUT_PAD)
    out = _mlp_tc(pooled, b1[perm].reshape(1, D), W2p, b2p)
    return out[:, :nout]


# speedup vs baseline: 5.0489x; 1.2160x over previous
import functools

import jax
import jax.numpy as jnp
from jax import lax
from jax.experimental import pallas as pl
from jax.experimental.pallas import tpu as pltpu
from jax.experimental.pallas import tpu_sc as plsc

B = 16384
H = 200
D = 64
V = 1_000_000
NC = 2
NS = 16
NW = NC * NS
BPW = B // NW
CRI = 32
NCH = BPW // CRI
H1, H2 = 104, 96

BN = 1024
CPB = 4 * BN
GA = (V + CPB - 1) // CPB
S4 = GA * BN
V2 = 4 * S4
WPR = D // 2

_PERM = (
    list(range(0, 16)) + list(range(32, 48))
    + list(range(16, 32)) + list(range(48, 64))
)


def _bf16_bits(x):
    u = lax.bitcast_convert_type(x, jnp.uint32)
    r = u + jnp.uint32(0x7FFF) + ((u >> 16) & jnp.uint32(1))
    return r & jnp.uint32(0xFFFF0000)


def _pack_body(t_ref, w_ref, o_ref):
    for k in range(4):
        rk = lax.dot_general(
            t_ref[:, k * BN:(k + 1) * BN], w_ref[:],
            (((0,), (0,)), ((), ())), preferred_element_type=jnp.float32)
        lo = _bf16_bits(rk[:, 0:WPR]) >> 16
        hi = _bf16_bits(rk[:, WPR:D])
        o_ref[:, WPR * k:WPR * (k + 1)] = lo | hi


def _pack_tc(tabT, W1s):
    return pl.pallas_call(
        _pack_body,
        grid=(GA,),
        in_specs=[
            pl.BlockSpec((D, CPB), lambda i: (0, i)),
            pl.BlockSpec((D, D), lambda i: (0, 0)),
        ],
        out_specs=pl.BlockSpec((BN, 4 * WPR), lambda i: (i, 0)),
        out_shape=jax.ShapeDtypeStruct((S4, 4 * WPR), jnp.uint32),
    )(tabT, W1s)


def _pool_sc(xf, tab):
    mesh = plsc.VectorSubcoreMesh(core_axis_name="c", subcore_axis_name="s")

    @functools.partial(
        pl.kernel,
        out_type=jax.ShapeDtypeStruct((B, D), jnp.float32),
        mesh=mesh,
        compiler_params=pltpu.CompilerParams(
            use_tc_tiling_on_sc=False, needs_layout_passes=False),
        scratch_types=[
            pltpu.VMEM((2 * CRI * H,), jnp.int32),
            pltpu.VMEM((4, H, WPR), jnp.uint32),
            pltpu.VMEM((BPW, D), jnp.float32),
            pltpu.SemaphoreType.DMA,
            pltpu.SemaphoreType.DMA,
            pltpu.SemaphoreType.DMA,
            pltpu.SemaphoreType.DMA,
            pltpu.SemaphoreType.DMA,
        ],
    )
    def kern(x_hbm, tab_hbm, out_hbm, idx_v, rows_v, pool_v,
             sem0, sem1, sem2, sem3, semi):
        sems = (sem0, sem1, sem2, sem3)
        wid = lax.axis_index("s") * NC + lax.axis_index("c")
        base = wid * BPW

        def start_gather(buf, sem, cbuf, rr):
            off = cbuf * CRI * H + rr * H
            pltpu.async_copy(
                tab_hbm.at[idx_v.at[pl.ds(off, H1)]],
                rows_v.at[buf, pl.ds(0, H1), :], sem)
            pltpu.async_copy(
                tab_hbm.at[idx_v.at[pl.ds(off + H1, H2)]],
                rows_v.at[buf, pl.ds(H1, H2), :], sem)

        def wait_gather(buf, sem):
            pltpu.make_async_copy(
                tab_hbm.at[pl.ds(0, H), :], rows_v.at[buf], sem).wait()

        def wait_idx():
            pltpu.make_async_copy(
                x_hbm.at[pl.ds(0, CRI * H)], idx_v.at[pl.ds(0, CRI * H)],
                semi).wait()

        hi_mask = jnp.full((16,), 0xFFFF0000, dtype=jnp.uint32)

        def accumulate(buf, row):
            def acc_body(j, acc):
                a0, a1, a2, a3 = acc
                for u in range(8):
                    jj = j * 8 + u
                    w0 = rows_v[buf, jj, pl.ds(0, 16)]
                    w1 = rows_v[buf, jj, pl.ds(16, 16)]
                    a0 = a0 + lax.bitcast_convert_type(
                        w0 << 16, jnp.float32)
                    a1 = a1 + lax.bitcast_convert_type(
                        w0 & hi_mask, jnp.float32)
                    a2 = a2 + lax.bitcast_convert_type(
                        w1 << 16, jnp.float32)
                    a3 = a3 + lax.bitcast_convert_type(
                        w1 & hi_mask, jnp.float32)
                return (a0, a1, a2, a3)

            zero = jnp.zeros((16,), jnp.float32)
            a0, a1, a2, a3 = lax.fori_loop(
                0, H // 8, acc_body, (zero, zero, zero, zero))
            pool_v[row, pl.ds(0, 16)] = a0
            pool_v[row, pl.ds(16, 16)] = a1
            pool_v[row, pl.ds(32, 16)] = a2
            pool_v[row, pl.ds(48, 16)] = a3

        pltpu.async_copy(
            x_hbm.at[pl.ds(base * H, CRI * H)],
            idx_v.at[pl.ds(0, CRI * H)], semi).wait()
        pltpu.async_copy(
            x_hbm.at[pl.ds((base + CRI) * H, CRI * H)],
            idx_v.at[pl.ds(CRI * H, CRI * H)], semi)
        start_gather(0, sem0, 0, 0)
        start_gather(1, sem1, 0, 1)
        start_gather(2, sem2, 0, 2)

        def body(i, _):
            r0 = 4 * i
            for u in range(4):
                wait_gather(u, sems[u])
                nxt = r0 + u + 3

                @pl.when(nxt < BPW)
                def _():
                    c_nxt = nxt // CRI
                    rr = lax.rem(nxt, CRI)
                    if u == 1:
                        @pl.when(rr == 0)
                        def _():
                            wait_idx()

                            @pl.when(c_nxt + 1 < NCH)
                            def _():
                                pltpu.async_copy(
                                    x_hbm.at[
                                        pl.ds((base + (c_nxt + 1) * CRI) * H,
                                              CRI * H)],
                                    idx_v.at[
                                        pl.ds(lax.rem(c_nxt + 1, 2) * CRI * H,
                                              CRI * H)], semi)

                    start_gather((u + 3) % 4, sems[(u + 3) % 4],
                                 lax.rem(c_nxt, 2), rr)

                accumulate(u, r0 + u)

            return 0

        lax.fori_loop(0, BPW // 4, body, 0)
        pltpu.sync_copy(pool_v, out_hbm.at[pl.ds(base, BPW), :])

    return kern(xf, tab)


BLK = 1024
NOUT_PAD = 128


def _mlp_body(p_ref, b1_ref, w2_ref, b2_ref, o_ref):
    h = jnp.maximum(p_ref[:] + b1_ref[:], 0.0)
    out = jnp.dot(h, w2_ref[:], preferred_element_type=jnp.float32)
    out = out + b2_ref[:]
    nrm = jnp.sqrt(jnp.sum(out * out, axis=-1, keepdims=True))
    o_ref[:] = out / jnp.maximum(nrm, 1e-12)


def _mlp_tc(pooled, b1r, W2p, b2p):
    return pl.pallas_call(
        _mlp_body,
        grid=(B // BLK,),
        in_specs=[
            pl.BlockSpec((BLK, D), lambda i: (i, 0)),
            pl.BlockSpec((1, D), lambda i: (0, 0)),
            pl.BlockSpec((D, NOUT_PAD), lambda i: (0, 0)),
            pl.BlockSpec((1, NOUT_PAD), lambda i: (0, 0)),
        ],
        out_specs=pl.BlockSpec((BLK, NOUT_PAD), lambda i: (i, 0)),
        out_shape=jax.ShapeDtypeStruct((B, NOUT_PAD), jnp.float32),
    )(pooled, b1r, W2p, b2p)


@jax.jit
def kernel(x, table, W1, b1, W2, b2):
    x = x.astype(jnp.int32)
    packed = _pack_tc(table.T, W1 * (1.0 / float(H)))
    tab2 = packed.reshape(V2, WPR)
    xr = ((x & ~4095) | ((x & 1023) << 2) | ((x >> 10) & 3)).reshape(-1)
    pooled = _pool_sc(xr, tab2)
    perm = jnp.asarray(_PERM, dtype=jnp.int32)
    nout = W2.shape[1]
    W2perm = W2[perm, :]
    W2p = jnp.pad(W2perm, ((0, 0), (0, NOUT_PAD - nout)))
    b2p = jnp.pad(b2, (0, NOUT_PAD - nout)).reshape(1, NOUT_PAD)
    out = _mlp_tc(pooled, b1[perm].reshape(1, D), W2p, b2p)
    return out[:, :nout]


# speedup vs baseline: 5.3969x; 1.0689x over previous
import functools

import jax
import jax.numpy as jnp
from jax import lax
from jax.experimental import pallas as pl
from jax.experimental.pallas import tpu as pltpu
from jax.experimental.pallas import tpu_sc as plsc

B = 16384
H = 200
D = 64
V = 1_000_000
NC = 2
NS = 16
NW = NC * NS
BPW = B // NW
CRI = 32
NCH = BPW // CRI
H1, H2 = 104, 96

BN = 2048
CPB = 4 * BN
GA = (V + CPB - 1) // CPB
S4 = GA * BN
V2 = 4 * S4
WPR = D // 2

_PERM = (
    list(range(0, 16)) + list(range(32, 48))
    + list(range(16, 32)) + list(range(48, 64))
)


def _bf16_bits(x):
    u = lax.bitcast_convert_type(x, jnp.uint32)
    r = u + jnp.uint32(0x7FFF) + ((u >> 16) & jnp.uint32(1))
    return r & jnp.uint32(0xFFFF0000)


def _pack_body(t_ref, w_ref, o_ref):
    for k in range(4):
        rk = lax.dot_general(
            t_ref[:, k * BN:(k + 1) * BN], w_ref[:],
            (((0,), (0,)), ((), ())), preferred_element_type=jnp.float32)
        lo = _bf16_bits(rk[:, 0:WPR]) >> 16
        hi = _bf16_bits(rk[:, WPR:D])
        o_ref[:, WPR * k:WPR * (k + 1)] = lo | hi


def _pack_tc(tabT, W1s):
    return pl.pallas_call(
        _pack_body,
        grid=(GA,),
        in_specs=[
            pl.BlockSpec((D, CPB), lambda i: (0, i)),
            pl.BlockSpec((D, D), lambda i: (0, 0)),
        ],
        out_specs=pl.BlockSpec((BN, 4 * WPR), lambda i: (i, 0)),
        out_shape=jax.ShapeDtypeStruct((S4, 4 * WPR), jnp.uint32),
    )(tabT, W1s)


def _pool_sc(xf, tab):
    mesh = plsc.VectorSubcoreMesh(core_axis_name="c", subcore_axis_name="s")

    @functools.partial(
        pl.kernel,
        out_type=jax.ShapeDtypeStruct((B, D), jnp.float32),
        mesh=mesh,
        compiler_params=pltpu.CompilerParams(
            use_tc_tiling_on_sc=False, needs_layout_passes=False),
        scratch_types=[
            pltpu.VMEM((2 * CRI * H,), jnp.int32),
            pltpu.VMEM((4, H, WPR), jnp.uint32),
            pltpu.VMEM((BPW, D), jnp.float32),
            pltpu.SemaphoreType.DMA,
            pltpu.SemaphoreType.DMA,
            pltpu.SemaphoreType.DMA,
            pltpu.SemaphoreType.DMA,
            pltpu.SemaphoreType.DMA,
        ],
    )
    def kern(x_hbm, tab_hbm, out_hbm, idx_v, rows_v, pool_v,
             sem0, sem1, sem2, sem3, semi):
        sems = (sem0, sem1, sem2, sem3)
        wid = lax.axis_index("s") * NC + lax.axis_index("c")
        base = wid * BPW

        def start_gather(buf, sem, cbuf, rr):
            off = cbuf * CRI * H + rr * H
            pltpu.async_copy(
                tab_hbm.at[idx_v.at[pl.ds(off, H1)]],
                rows_v.at[buf, pl.ds(0, H1), :], sem)
            pltpu.async_copy(
                tab_hbm.at[idx_v.at[pl.ds(off + H1, H2)]],
                rows_v.at[buf, pl.ds(H1, H2), :], sem)

        def wait_gather(buf, sem):
            pltpu.make_async_copy(
                tab_hbm.at[pl.ds(0, H), :], rows_v.at[buf], sem).wait()

        def wait_idx():
            pltpu.make_async_copy(
                x_hbm.at[pl.ds(0, CRI * H)], idx_v.at[pl.ds(0, CRI * H)],
                semi).wait()

        hi_mask = jnp.full((16,), 0xFFFF0000, dtype=jnp.uint32)

        def accumulate(buf, row):
            def acc_body(j, acc):
                a0, a1, a2, a3 = acc
                for u in range(8):
                    jj = j * 8 + u
                    w0 = rows_v[buf, jj, pl.ds(0, 16)]
                    w1 = rows_v[buf, jj, pl.ds(16, 16)]
                    a0 = a0 + lax.bitcast_convert_type(
                        w0 << 16, jnp.float32)
                    a1 = a1 + lax.bitcast_convert_type(
                        w0 & hi_mask, jnp.float32)
                    a2 = a2 + lax.bitcast_convert_type(
                        w1 << 16, jnp.float32)
                    a3 = a3 + lax.bitcast_convert_type(
                        w1 & hi_mask, jnp.float32)
                return (a0, a1, a2, a3)

            zero = jnp.zeros((16,), jnp.float32)
            a0, a1, a2, a3 = lax.fori_loop(
                0, H // 8, acc_body, (zero, zero, zero, zero))
            pool_v[row, pl.ds(0, 16)] = a0
            pool_v[row, pl.ds(16, 16)] = a1
            pool_v[row, pl.ds(32, 16)] = a2
            pool_v[row, pl.ds(48, 16)] = a3

        pltpu.async_copy(
            x_hbm.at[pl.ds(base * H, CRI * H)],
            idx_v.at[pl.ds(0, CRI * H)], semi).wait()
        pltpu.async_copy(
            x_hbm.at[pl.ds((base + CRI) * H, CRI * H)],
            idx_v.at[pl.ds(CRI * H, CRI * H)], semi)
        start_gather(0, sem0, 0, 0)
        start_gather(1, sem1, 0, 1)
        start_gather(2, sem2, 0, 2)

        def body(i, _):
            r0 = 4 * i
            for u in range(4):
                wait_gather(u, sems[u])
                nxt = r0 + u + 3

                @pl.when(nxt < BPW)
                def _():
                    c_nxt = nxt // CRI
                    rr = lax.rem(nxt, CRI)
                    if u == 1:
                        @pl.when(rr == 0)
                        def _():
                            wait_idx()

                            @pl.when(c_nxt + 1 < NCH)
                            def _():
                                pltpu.async_copy(
                                    x_hbm.at[
                                        pl.ds((base + (c_nxt + 1) * CRI) * H,
                                              CRI * H)],
                                    idx_v.at[
                                        pl.ds(lax.rem(c_nxt + 1, 2) * CRI * H,
                                              CRI * H)], semi)

                    start_gather((u + 3) % 4, sems[(u + 3) % 4],
                                 lax.rem(c_nxt, 2), rr)

                accumulate(u, r0 + u)

            return 0

        lax.fori_loop(0, BPW // 4, body, 0)
        pltpu.sync_copy(pool_v, out_hbm.at[pl.ds(base, BPW), :])

    return kern(xf, tab)


BLK = 1024
NOUT_PAD = 128


def _mlp_body(p_ref, b1_ref, w2_ref, b2_ref, o_ref):
    h = jnp.maximum(p_ref[:] + b1_ref[:], 0.0)
    out = jnp.dot(h, w2_ref[:], preferred_element_type=jnp.float32)
    out = out + b2_ref[:]
    nrm = jnp.sqrt(jnp.sum(out * out, axis=-1, keepdims=True))
    o_ref[:] = out / jnp.maximum(nrm, 1e-12)


def _mlp_tc(pooled, b1r, W2p, b2p):
    return pl.pallas_call(
        _mlp_body,
        grid=(B // BLK,),
        in_specs=[
            pl.BlockSpec((BLK, D), lambda i: (i, 0)),
            pl.BlockSpec((1, D), lambda i: (0, 0)),
            pl.BlockSpec((D, NOUT_PAD), lambda i: (0, 0)),
            pl.BlockSpec((1, NOUT_PAD), lambda i: (0, 0)),
        ],
        out_specs=pl.BlockSpec((BLK, NOUT_PAD), lambda i: (i, 0)),
        out_shape=jax.ShapeDtypeStruct((B, NOUT_PAD), jnp.float32),
    )(pooled, b1r, W2p, b2p)


@jax.jit
def kernel(x, table, W1, b1, W2, b2):
    x = x.astype(jnp.int32)
    packed = _pack_tc(table.T, W1 * (1.0 / float(H)))
    tab2 = packed.reshape(V2, WPR)
    bnlog = BN.bit_length() - 1
    xr = ((x & ~(CPB - 1)) | ((x & (BN - 1)) << 2)
          | ((x >> bnlog) & 3)).reshape(-1)
    pooled = _pool_sc(xr, tab2)
    perm = jnp.asarray(_PERM, dtype=jnp.int32)
    nout = W2.shape[1]
    W2perm = W2[perm, :]
    W2p = jnp.pad(W2perm, ((0, 0), (0, NOUT_PAD - nout)))
    b2p = jnp.pad(b2, (0, NOUT_PAD - nout)).reshape(1, NOUT_PAD)
    out = _mlp_tc(pooled, b1[perm].reshape(1, D), W2p, b2p)
    return out[:, :nout]


# speedup vs baseline: 5.4606x; 1.0118x over previous
import functools

import jax
import jax.numpy as jnp
from jax import lax
from jax.experimental import pallas as pl
from jax.experimental.pallas import tpu as pltpu
from jax.experimental.pallas import tpu_sc as plsc

B = 16384
H = 200
D = 64
V = 1_000_000
NC = 2
NS = 16
NW = NC * NS
BPW = B // NW
CRI = 32
NCH = BPW // CRI
H1, H2 = 104, 96

BN = 4096
CPB = 4 * BN
GA = (V + CPB - 1) // CPB
S4 = GA * BN
V2 = 4 * S4
WPR = D // 2

_PERM = (
    list(range(0, 16)) + list(range(32, 48))
    + list(range(16, 32)) + list(range(48, 64))
)


def _bf16_bits(x):
    u = lax.bitcast_convert_type(x, jnp.uint32)
    r = u + jnp.uint32(0x7FFF) + ((u >> 16) & jnp.uint32(1))
    return r & jnp.uint32(0xFFFF0000)


def _pack_body(t_ref, w_ref, o_ref):
    for k in range(4):
        rk = lax.dot_general(
            t_ref[:, k * BN:(k + 1) * BN], w_ref[:],
            (((0,), (0,)), ((), ())), preferred_element_type=jnp.float32)
        lo = _bf16_bits(rk[:, 0:WPR]) >> 16
        hi = _bf16_bits(rk[:, WPR:D])
        o_ref[:, WPR * k:WPR * (k + 1)] = lo | hi


def _pack_tc(tabT, W1s):
    return pl.pallas_call(
        _pack_body,
        grid=(GA,),
        in_specs=[
            pl.BlockSpec((D, CPB), lambda i: (0, i)),
            pl.BlockSpec((D, D), lambda i: (0, 0)),
        ],
        out_specs=pl.BlockSpec((BN, 4 * WPR), lambda i: (i, 0)),
        out_shape=jax.ShapeDtypeStruct((S4, 4 * WPR), jnp.uint32),
    )(tabT, W1s)


def _pool_sc(xf, tab):
    mesh = plsc.VectorSubcoreMesh(core_axis_name="c", subcore_axis_name="s")

    @functools.partial(
        pl.kernel,
        out_type=jax.ShapeDtypeStruct((B, D), jnp.float32),
        mesh=mesh,
        compiler_params=pltpu.CompilerParams(
            use_tc_tiling_on_sc=False, needs_layout_passes=False),
        scratch_types=[
            pltpu.VMEM((2 * CRI * H,), jnp.int32),
            pltpu.VMEM((4, H, WPR), jnp.uint32),
            pltpu.VMEM((BPW, D), jnp.float32),
            pltpu.SemaphoreType.DMA,
            pltpu.SemaphoreType.DMA,
            pltpu.SemaphoreType.DMA,
            pltpu.SemaphoreType.DMA,
            pltpu.SemaphoreType.DMA,
        ],
    )
    def kern(x_hbm, tab_hbm, out_hbm, idx_v, rows_v, pool_v,
             sem0, sem1, sem2, sem3, semi):
        sems = (sem0, sem1, sem2, sem3)
        wid = lax.axis_index("s") * NC + lax.axis_index("c")
        base = wid * BPW

        def start_gather(buf, sem, cbuf, rr):
            off = cbuf * CRI * H + rr * H
            pltpu.async_copy(
                tab_hbm.at[idx_v.at[pl.ds(off, H1)]],
                rows_v.at[buf, pl.ds(0, H1), :], sem)
            pltpu.async_copy(
                tab_hbm.at[idx_v.at[pl.ds(off + H1, H2)]],
                rows_v.at[buf, pl.ds(H1, H2), :], sem)

        def wait_gather(buf, sem):
            pltpu.make_async_copy(
                tab_hbm.at[pl.ds(0, H), :], rows_v.at[buf], sem).wait()

        def wait_idx():
            pltpu.make_async_copy(
                x_hbm.at[pl.ds(0, CRI * H)], idx_v.at[pl.ds(0, CRI * H)],
                semi).wait()

        hi_mask = jnp.full((16,), 0xFFFF0000, dtype=jnp.uint32)

        def accumulate(buf, row):
            def acc_body(j, acc):
                a0, a1, a2, a3 = acc
                for u in range(8):
                    jj = j * 8 + u
                    w0 = rows_v[buf, jj, pl.ds(0, 16)]
                    w1 = rows_v[buf, jj, pl.ds(16, 16)]
                    a0 = a0 + lax.bitcast_convert_type(
                        w0 << 16, jnp.float32)
                    a1 = a1 + lax.bitcast_convert_type(
                        w0 & hi_mask, jnp.float32)
                    a2 = a2 + lax.bitcast_convert_type(
                        w1 << 16, jnp.float32)
                    a3 = a3 + lax.bitcast_convert_type(
                        w1 & hi_mask, jnp.float32)
                return (a0, a1, a2, a3)

            zero = jnp.zeros((16,), jnp.float32)
            a0, a1, a2, a3 = lax.fori_loop(
                0, H // 8, acc_body, (zero, zero, zero, zero))
            pool_v[row, pl.ds(0, 16)] = a0
            pool_v[row, pl.ds(16, 16)] = a1
            pool_v[row, pl.ds(32, 16)] = a2
            pool_v[row, pl.ds(48, 16)] = a3

        pltpu.async_copy(
            x_hbm.at[pl.ds(base * H, CRI * H)],
            idx_v.at[pl.ds(0, CRI * H)], semi).wait()
        pltpu.async_copy(
            x_hbm.at[pl.ds((base + CRI) * H, CRI * H)],
            idx_v.at[pl.ds(CRI * H, CRI * H)], semi)
        start_gather(0, sem0, 0, 0)
        start_gather(1, sem1, 0, 1)
        start_gather(2, sem2, 0, 2)

        def body(i, _):
            r0 = 4 * i
            for u in range(4):
                wait_gather(u, sems[u])
                nxt = r0 + u + 3

                @pl.when(nxt < BPW)
                def _():
                    c_nxt = nxt // CRI
                    rr = lax.rem(nxt, CRI)
                    if u == 1:
                        @pl.when(rr == 0)
                        def _():
                            wait_idx()

                            @pl.when(c_nxt + 1 < NCH)
                            def _():
                                pltpu.async_copy(
                                    x_hbm.at[
                                        pl.ds((base + (c_nxt + 1) * CRI) * H,
                                              CRI * H)],
                                    idx_v.at[
                                        pl.ds(lax.rem(c_nxt + 1, 2) * CRI * H,
                                              CRI * H)], semi)

                    start_gather((u + 3) % 4, sems[(u + 3) % 4],
                                 lax.rem(c_nxt, 2), rr)

                accumulate(u, r0 + u)

            return 0

        lax.fori_loop(0, BPW // 4, body, 0)
        pltpu.sync_copy(pool_v, out_hbm.at[pl.ds(base, BPW), :])

    return kern(xf, tab)


BLK = 1024
NOUT_PAD = 128


def _mlp_body(p_ref, b1_ref, w2_ref, b2_ref, o_ref):
    h = jnp.maximum(p_ref[:] + b1_ref[:], 0.0)
    out = jnp.dot(h, w2_ref[:], preferred_element_type=jnp.float32)
    out = out + b2_ref[:]
    nrm = jnp.sqrt(jnp.sum(out * out, axis=-1, keepdims=True))
    o_ref[:] = out / jnp.maximum(nrm, 1e-12)


def _mlp_tc(pooled, b1r, W2p, b2p):
    return pl.pallas_call(
        _mlp_body,
        grid=(B // BLK,),
        in_specs=[
            pl.BlockSpec((BLK, D), lambda i: (i, 0)),
            pl.BlockSpec((1, D), lambda i: (0, 0)),
            pl.BlockSpec((D, NOUT_PAD), lambda i: (0, 0)),
            pl.BlockSpec((1, NOUT_PAD), lambda i: (0, 0)),
        ],
        out_specs=pl.BlockSpec((BLK, NOUT_PAD), lambda i: (i, 0)),
        out_shape=jax.ShapeDtypeStruct((B, NOUT_PAD), jnp.float32),
    )(pooled, b1r, W2p, b2p)


@jax.jit
def kernel(x, table, W1, b1, W2, b2):
    x = x.astype(jnp.int32)
    packed = _pack_tc(table.T, W1 * (1.0 / float(H)))
    tab2 = packed.reshape(V2, WPR)
    bnlog = BN.bit_length() - 1
    xr = ((x & ~(CPB - 1)) | ((x & (BN - 1)) << 2)
          | ((x >> bnlog) & 3)).reshape(-1)
    pooled = _pool_sc(xr, tab2)
    perm = jnp.asarray(_PERM, dtype=jnp.int32)
    nout = W2.shape[1]
    W2perm = W2[perm, :]
    W2p = jnp.pad(W2perm, ((0, 0), (0, NOUT_PAD - nout)))
    b2p = jnp.pad(b2, (0, NOUT_PAD - nout)).reshape(1, NOUT_PAD)
    out = _mlp_tc(pooled, b1[perm].reshape(1, D), W2p, b2p)
    return out[:, :nout]


# speedup vs baseline: 5.5073x; 1.0086x over previous
import functools

import jax
import jax.numpy as jnp
from jax import lax
from jax.experimental import pallas as pl
from jax.experimental.pallas import tpu as pltpu
from jax.experimental.pallas import tpu_sc as plsc

B = 16384
H = 200
D = 64
V = 1_000_000
NC = 2
NS = 16
NW = NC * NS
BPW = B // NW
CRI = 32
NCH = BPW // CRI
H1, H2 = 104, 96

BN = 8192
CPB = 4 * BN
GA = (V + CPB - 1) // CPB
S4 = GA * BN
V2 = 4 * S4
WPR = D // 2

_PERM = (
    list(range(0, 16)) + list(range(32, 48))
    + list(range(16, 32)) + list(range(48, 64))
)


def _bf16_bits(x):
    u = lax.bitcast_convert_type(x, jnp.uint32)
    r = u + jnp.uint32(0x7FFF) + ((u >> 16) & jnp.uint32(1))
    return r & jnp.uint32(0xFFFF0000)


def _pack_body(t_ref, w_ref, o_ref):
    for k in range(4):
        rk = lax.dot_general(
            t_ref[:, k * BN:(k + 1) * BN], w_ref[:],
            (((0,), (0,)), ((), ())), preferred_element_type=jnp.float32)
        lo = _bf16_bits(rk[:, 0:WPR]) >> 16
        hi = _bf16_bits(rk[:, WPR:D])
        o_ref[:, WPR * k:WPR * (k + 1)] = lo | hi


def _pack_tc(tabT, W1s):
    return pl.pallas_call(
        _pack_body,
        grid=(GA,),
        in_specs=[
            pl.BlockSpec((D, CPB), lambda i: (0, i)),
            pl.BlockSpec((D, D), lambda i: (0, 0)),
        ],
        out_specs=pl.BlockSpec((BN, 4 * WPR), lambda i: (i, 0)),
        out_shape=jax.ShapeDtypeStruct((S4, 4 * WPR), jnp.uint32),
    )(tabT, W1s)


def _pool_sc(xf, tab):
    mesh = plsc.VectorSubcoreMesh(core_axis_name="c", subcore_axis_name="s")

    @functools.partial(
        pl.kernel,
        out_type=jax.ShapeDtypeStruct((B, D), jnp.float32),
        mesh=mesh,
        compiler_params=pltpu.CompilerParams(
            use_tc_tiling_on_sc=False, needs_layout_passes=False),
        scratch_types=[
            pltpu.VMEM((2 * CRI * H,), jnp.int32),
            pltpu.VMEM((4, H, WPR), jnp.uint32),
            pltpu.VMEM((BPW, D), jnp.float32),
            pltpu.SemaphoreType.DMA,
            pltpu.SemaphoreType.DMA,
            pltpu.SemaphoreType.DMA,
            pltpu.SemaphoreType.DMA,
            pltpu.SemaphoreType.DMA,
        ],
    )
    def kern(x_hbm, tab_hbm, out_hbm, idx_v, rows_v, pool_v,
             sem0, sem1, sem2, sem3, semi):
        sems = (sem0, sem1, sem2, sem3)
        wid = lax.axis_index("s") * NC + lax.axis_index("c")
        base = wid * BPW

        def start_gather(buf, sem, cbuf, rr):
            off = cbuf * CRI * H + rr * H
            pltpu.async_copy(
                tab_hbm.at[idx_v.at[pl.ds(off, H1)]],
                rows_v.at[buf, pl.ds(0, H1), :], sem)
            pltpu.async_copy(
                tab_hbm.at[idx_v.at[pl.ds(off + H1, H2)]],
                rows_v.at[buf, pl.ds(H1, H2), :], sem)

        def wait_gather(buf, sem):
            pltpu.make_async_copy(
                tab_hbm.at[pl.ds(0, H), :], rows_v.at[buf], sem).wait()

        def wait_idx():
            pltpu.make_async_copy(
                x_hbm.at[pl.ds(0, CRI * H)], idx_v.at[pl.ds(0, CRI * H)],
                semi).wait()

        hi_mask = jnp.full((16,), 0xFFFF0000, dtype=jnp.uint32)

        def accumulate(buf, row):
            def acc_body(j, acc):
                a0, a1, a2, a3 = acc
                for u in range(8):
                    jj = j * 8 + u
                    w0 = rows_v[buf, jj, pl.ds(0, 16)]
                    w1 = rows_v[buf, jj, pl.ds(16, 16)]
                    a0 = a0 + lax.bitcast_convert_type(
                        w0 << 16, jnp.float32)
                    a1 = a1 + lax.bitcast_convert_type(
                        w0 & hi_mask, jnp.float32)
                    a2 = a2 + lax.bitcast_convert_type(
                        w1 << 16, jnp.float32)
                    a3 = a3 + lax.bitcast_convert_type(
                        w1 & hi_mask, jnp.float32)
                return (a0, a1, a2, a3)

            zero = jnp.zeros((16,), jnp.float32)
            a0, a1, a2, a3 = lax.fori_loop(
                0, H // 8, acc_body, (zero, zero, zero, zero))
            pool_v[row, pl.ds(0, 16)] = a0
            pool_v[row, pl.ds(16, 16)] = a1
            pool_v[row, pl.ds(32, 16)] = a2
            pool_v[row, pl.ds(48, 16)] = a3

        pltpu.async_copy(
            x_hbm.at[pl.ds(base * H, CRI * H)],
            idx_v.at[pl.ds(0, CRI * H)], semi).wait()
        pltpu.async_copy(
            x_hbm.at[pl.ds((base + CRI) * H, CRI * H)],
            idx_v.at[pl.ds(CRI * H, CRI * H)], semi)
        start_gather(0, sem0, 0, 0)
        start_gather(1, sem1, 0, 1)
        start_gather(2, sem2, 0, 2)

        def body(i, _):
            r0 = 4 * i
            for u in range(4):
                wait_gather(u, sems[u])
                nxt = r0 + u + 3

                @pl.when(nxt < BPW)
                def _():
                    c_nxt = nxt // CRI
                    rr = lax.rem(nxt, CRI)
                    if u == 1:
                        @pl.when(rr == 0)
                        def _():
                            wait_idx()

                            @pl.when(c_nxt + 1 < NCH)
                            def _():
                                pltpu.async_copy(
                                    x_hbm.at[
                                        pl.ds((base + (c_nxt + 1) * CRI) * H,
                                              CRI * H)],
                                    idx_v.at[
                                        pl.ds(lax.rem(c_nxt + 1, 2) * CRI * H,
                                              CRI * H)], semi)

                    start_gather((u + 3) % 4, sems[(u + 3) % 4],
                                 lax.rem(c_nxt, 2), rr)

                accumulate(u, r0 + u)

            return 0

        lax.fori_loop(0, BPW // 4, body, 0)
        pltpu.sync_copy(pool_v, out_hbm.at[pl.ds(base, BPW), :])

    return kern(xf, tab)


BLK = 1024
NOUT_PAD = 128


def _mlp_body(p_ref, b1_ref, w2_ref, b2_ref, o_ref):
    h = jnp.maximum(p_ref[:] + b1_ref[:], 0.0)
    out = jnp.dot(h, w2_ref[:], preferred_element_type=jnp.float32)
    out = out + b2_ref[:]
    nrm = jnp.sqrt(jnp.sum(out * out, axis=-1, keepdims=True))
    o_ref[:] = out / jnp.maximum(nrm, 1e-12)


def _mlp_tc(pooled, b1r, W2p, b2p):
    return pl.pallas_call(
        _mlp_body,
        grid=(B // BLK,),
        in_specs=[
            pl.BlockSpec((BLK, D), lambda i: (i, 0)),
            pl.BlockSpec((1, D), lambda i: (0, 0)),
            pl.BlockSpec((D, NOUT_PAD), lambda i: (0, 0)),
            pl.BlockSpec((1, NOUT_PAD), lambda i: (0, 0)),
        ],
        out_specs=pl.BlockSpec((BLK, NOUT_PAD), lambda i: (i, 0)),
        out_shape=jax.ShapeDtypeStruct((B, NOUT_PAD), jnp.float32),
    )(pooled, b1r, W2p, b2p)


@jax.jit
def kernel(x, table, W1, b1, W2, b2):
    x = x.astype(jnp.int32)
    packed = _pack_tc(table.T, W1 * (1.0 / float(H)))
    tab2 = packed.reshape(V2, WPR)
    bnlog = BN.bit_length() - 1
    xr = ((x & ~(CPB - 1)) | ((x & (BN - 1)) << 2)
          | ((x >> bnlog) & 3)).reshape(-1)
    pooled = _pool_sc(xr, tab2)
    perm = jnp.asarray(_PERM, dtype=jnp.int32)
    nout = W2.shape[1]
    W2perm = W2[perm, :]
    W2p = jnp.pad(W2perm, ((0, 0), (0, NOUT_PAD - nout)))
    b2p = jnp.pad(b2, (0, NOUT_PAD - nout)).reshape(1, NOUT_PAD)
    out = _mlp_tc(pooled, b1[perm].reshape(1, D), W2p, b2p)
    return out[:, :nout]
